# baseline (speedup 1.0000x reference)
import functools

import jax
import jax.numpy as jnp
from jax import lax
from jax.experimental import pallas as pl
from jax.experimental.pallas import tpu as pltpu
from jax.experimental.pallas import tpu_sc as plsc

N = 10000
E = 320000
R = 3
IN = 128
H = 64
C = 5

NC = 2
NS = 16
L = 16
NW = NC * NS

EK = 64
NT = 160
EP = NT * NW * EK
GSL = 8
TRASH = 128

RN = R * N
DCH = 200
NDC = RN // DCH
DT = -(-NDC // NS)

_MESH = plsc.VectorSubcoreMesh(core_axis_name="c", subcore_axis_name="s")
_SC_PARAMS = pltpu.CompilerParams(use_tc_tiling_on_sc=False)


def _sc_aggregate(y_flat, zeros, pkg, pkw):
    @functools.partial(
        pl.kernel,
        out_type=jax.ShapeDtypeStruct((NC, RN, H), jnp.float32),
        mesh=_MESH,
        scratch_types=[
            pltpu.VMEM_SHARED((RN + TRASH, H), jnp.float32),
            pltpu.VMEM((GSL, EK), jnp.int32),
            pltpu.VMEM((GSL, EK), jnp.int32),
            pltpu.VMEM((EK, H), jnp.float32),
            pltpu.VMEM((EK, H), jnp.float32),
            pltpu.SemaphoreType.DMA,
            pltpu.SemaphoreType.DMA,
            pltpu.SemaphoreType.DMA,
            pltpu.SemaphoreType.DMA,
        ],
        compiler_params=_SC_PARAMS,
    )
    def k(y_hbm, z_hbm, pkg_hbm, pkw_hbm, out_hbm, acc, gslab, wslab,
          rowsa, rowsb, semga, semgb, semsa, semsb):
        cid = lax.axis_index("c")
        sid = lax.axis_index("s")
        wid = sid * NC + cid

        @pl.loop(0, DT)
        def _(t):
            ci = sid + t * NS

            @pl.when(ci < NDC)
            def _():
                pltpu.sync_copy(z_hbm, acc.at[pl.ds(ci * DCH, DCH)])

        plsc.subcore_barrier()

        @pl.loop(0, NT // GSL)
        def _(g):
            pltpu.sync_copy(pkg_hbm.at[pl.ds(g * GSL, GSL), wid], gslab)
            pltpu.sync_copy(pkw_hbm.at[pl.ds(g * GSL, GSL), wid], wslab)
            for j in range(0, GSL, 2):
                ga = pltpu.async_copy(y_hbm.at[gslab.at[j]], rowsa,
                                      semga)
                gb = pltpu.async_copy(y_hbm.at[gslab.at[j + 1]], rowsb,
                                      semgb)
                ga.wait()
                sa = pltpu.async_copy(rowsa, acc.at[wslab.at[j]], semsa,
                                      add=True)
                gb.wait()
                sb = pltpu.async_copy(rowsb, acc.at[wslab.at[j + 1]],
                                      semsb, add=True)
                sa.wait()
                sb.wait()

        plsc.subcore_barrier()

        @pl.loop(0, DT)
        def _(t):
            ci = sid + t * NS

            @pl.when(ci < NDC)
            def _():
                pltpu.sync_copy(acc.at[pl.ds(ci * DCH, DCH)],
                                out_hbm.at[cid, pl.ds(ci * DCH, DCH)])

    return k(y_flat, zeros, pkg, pkw)


def _sc_counts(zeros, pkw):
    @functools.partial(
        pl.kernel,
        out_type=jax.ShapeDtypeStruct((NC, RN, L), jnp.float32),
        mesh=_MESH,
        scratch_types=[
            pltpu.VMEM_SHARED((RN + TRASH, L), jnp.float32),
            pltpu.VMEM((GSL, EK), jnp.int32),
            pltpu.VMEM((EK, L), jnp.float32),
        ],
        compiler_params=_SC_PARAMS,
    )
    def k(z_hbm, pkw_hbm, out_hbm, acc, slab, obuf):
        cid = lax.axis_index("c")
        sid = lax.axis_index("s")
        wid = sid * NC + cid

        onehot = jnp.where(lax.iota(jnp.int32, L) == 0,
                           jnp.float32(1.0), jnp.float32(0.0))

        @pl.loop(0, EK)
        def _(i):
            obuf[i, :] = onehot

        @pl.loop(0, DT)
        def _(t):
            ci = sid + t * NS

            @pl.when(ci < NDC)
            def _():
                pltpu.sync_copy(z_hbm, acc.at[pl.ds(ci * DCH, DCH)])

        plsc.subcore_barrier()

        @pl.loop(0, NT // GSL)
        def _(g):
            pltpu.sync_copy(pkw_hbm.at[pl.ds(g * GSL, GSL), wid], slab)
            for j in range(GSL):
                pltpu.sync_copy(obuf, acc.at[slab.at[j]], add=True)

        plsc.subcore_barrier()

        @pl.loop(0, DT)
        def _(t):
            ci = sid + t * NS

            @pl.when(ci < NDC)
            def _():
                pltpu.sync_copy(acc.at[pl.ds(ci * DCH, DCH)],
                                out_hbm.at[cid, pl.ds(ci * DCH, DCH)])

    return k(zeros, pkw)


_TB = 16


def _tc_pack(src_p, dst_p, typ_p):
    def body(s_ref, d_ref, t_ref, pkg_ref, pkw_ref):
        tn = t_ref[...] * N
        pkg_ref[...] = tn + s_ref[...]
        pkw_ref[...] = tn + d_ref[...]

    return pl.pallas_call(
        body,
        grid=(NT // _TB,),
        in_specs=[
            pl.BlockSpec((_TB, NW, EK), lambda i: (i, 0, 0)),
            pl.BlockSpec((_TB, NW, EK), lambda i: (i, 0, 0)),
            pl.BlockSpec((_TB, NW, EK), lambda i: (i, 0, 0)),
        ],
        out_specs=[
            pl.BlockSpec((_TB, NW, EK), lambda i: (i, 0, 0)),
            pl.BlockSpec((_TB, NW, EK), lambda i: (i, 0, 0)),
        ],
        out_shape=[
            jax.ShapeDtypeStruct((NT, NW, EK), jnp.int32),
            jax.ShapeDtypeStruct((NT, NW, EK), jnp.int32),
        ],
    )(src_p, dst_p, typ_p)


_NB = 2000


def _dot(a, b):
    return jax.lax.dot_general(a, b, (((1,), (0,)), ((), ())),
                               precision=lax.Precision.HIGHEST,
                               preferred_element_type=jnp.float32)


def _dense1(x, W_rel1, W_root1, b1):
    def body(x_ref, wr_ref, wroot_ref, b_ref, y_ref, root_ref):
        xb = x_ref[...]
        for r in range(R):
            y_ref[r] = _dot(xb, wr_ref[r])
        root_ref[...] = _dot(xb, wroot_ref[...]) + b_ref[...]

    grid = (N // _NB,)
    y, root = pl.pallas_call(
        body,
        grid=grid,
        in_specs=[
            pl.BlockSpec((_NB, IN), lambda i: (i, 0)),
            pl.BlockSpec((R, IN, H), lambda i: (0, 0, 0)),
            pl.BlockSpec((IN, H), lambda i: (0, 0)),
            pl.BlockSpec((1, H), lambda i: (0, 0)),
        ],
        out_specs=[
            pl.BlockSpec((R, _NB, H), lambda i: (0, i, 0)),
            pl.BlockSpec((_NB, H), lambda i: (i, 0)),
        ],
        out_shape=[
            jax.ShapeDtypeStruct((R, N, H), jnp.float32),
            jax.ShapeDtypeStruct((N, H), jnp.float32),
        ],
    )(x, W_rel1, W_root1, b1.reshape(1, H))
    return y.reshape(RN, H), root


def _combine(root_blk, p_ref, cnt_ref):
    h = root_blk
    for r in range(R):
        s = p_ref[0, r] + p_ref[1, r]
        cnt = cnt_ref[0, r, :, 0:1] + cnt_ref[1, r, :, 0:1]
        h = h + s * (1.0 / jnp.maximum(cnt, 1.0))
    return jnp.maximum(h, 0.0)


def _dense2(root1, p1, cnt, W_rel2, W_root2, b2):
    def body(root_ref, p_ref, cnt_ref, wr_ref, wroot_ref, b_ref,
             y_ref, root2_ref):
        h = _combine(root_ref[...], p_ref, cnt_ref)
        for r in range(R):
            y_ref[r] = _dot(h, wr_ref[r])
        root2_ref[...] = _dot(h, wroot_ref[...]) + b_ref[...]

    grid = (N // _NB,)
    y, root2 = pl.pallas_call(
        body,
        grid=grid,
        in_specs=[
            pl.BlockSpec((_NB, H), lambda i: (i, 0)),
            pl.BlockSpec((NC, R, _NB, H), lambda i: (0, 0, i, 0)),
            pl.BlockSpec((NC, R, _NB, L), lambda i: (0, 0, i, 0)),
            pl.BlockSpec((R, H, H), lambda i: (0, 0, 0)),
            pl.BlockSpec((H, H), lambda i: (0, 0)),
            pl.BlockSpec((1, H), lambda i: (0, 0)),
        ],
        out_specs=[
            pl.BlockSpec((R, _NB, H), lambda i: (0, i, 0)),
            pl.BlockSpec((_NB, H), lambda i: (i, 0)),
        ],
        out_shape=[
            jax.ShapeDtypeStruct((R, N, H), jnp.float32),
            jax.ShapeDtypeStruct((N, H), jnp.float32),
        ],
    )(root1, p1.reshape(NC, R, N, H), cnt.reshape(NC, R, N, L),
      W_rel2, W_root2, b2.reshape(1, H))
    return y.reshape(RN, H), root2


def _final(root2, p2, cnt, Wc, bc):
    def body(root_ref, p_ref, cnt_ref, wc_ref, bc_ref, out_ref):
        h = _combine(root_ref[...], p_ref, cnt_ref)
        out_ref[...] = _dot(h, wc_ref[...]) + bc_ref[...]

    grid = (N // _NB,)
    return pl.pallas_call(
        body,
        grid=grid,
        in_specs=[
            pl.BlockSpec((_NB, H), lambda i: (i, 0)),
            pl.BlockSpec((NC, R, _NB, H), lambda i: (0, 0, i, 0)),
            pl.BlockSpec((NC, R, _NB, L), lambda i: (0, 0, i, 0)),
            pl.BlockSpec((H, C), lambda i: (0, 0)),
            pl.BlockSpec((1, C), lambda i: (0, 0)),
        ],
        out_specs=pl.BlockSpec((_NB, C), lambda i: (i, 0)),
        out_shape=jax.ShapeDtypeStruct((N, C), jnp.float32),
    )(root2, p2.reshape(NC, R, N, H), cnt.reshape(NC, R, N, L),
      Wc, bc.reshape(1, C))


def kernel(x, edge_index, edge_type, W_rel1, W_root1, b1,
           W_rel2, W_root2, b2, Wc, bc):
    src = edge_index[0]
    dst = edge_index[1]
    npad = EP - E
    spread = jnp.arange(npad, dtype=jnp.int32)
    src_p = jnp.concatenate(
        [src, spread % N]).reshape(NT, NW, EK)
    dst_p = jnp.concatenate(
        [dst, N + spread % TRASH]).reshape(NT, NW, EK)
    typ_p = jnp.concatenate(
        [edge_type, jnp.full((npad,), R - 1, jnp.int32)]).reshape(
            NT, NW, EK)
    pkg, pkw = _tc_pack(src_p, dst_p, typ_p)
    zeros = jnp.zeros((DCH, H), jnp.float32)
    cnt = _sc_counts(jnp.zeros((DCH, L), jnp.float32), pkw)
    y1, root1 = _dense1(x, W_rel1, W_root1, b1)
    p1 = _sc_aggregate(y1, zeros, pkg, pkw)
    y2, root2 = _dense2(root1, p1, cnt, W_rel2, W_root2, b2)
    p2 = _sc_aggregate(y2, zeros, pkg, pkw)
    return _final(root2, p2, cnt, Wc, bc)


# speedup vs baseline: 1.0810x; 1.0810x over previous
import functools

import jax
import jax.numpy as jnp
from jax import lax
from jax.experimental import pallas as pl
from jax.experimental.pallas import tpu as pltpu
from jax.experimental.pallas import tpu_sc as plsc

N = 10000
E = 320000
R = 3
IN = 128
H = 64
C = 5

NC = 2
NS = 16
L = 16
NW = NC * NS

EK = 64
NT = 160
EP = NT * NW * EK
GSL = 16
TRASH = 128

RN = R * N
DCH = 200
NDC = RN // DCH
DT = -(-NDC // NS)

_MESH = plsc.VectorSubcoreMesh(core_axis_name="c", subcore_axis_name="s")
_SC_PARAMS = pltpu.CompilerParams(use_tc_tiling_on_sc=False)


def _sc_aggregate(y_flat, zeros, pkg, pkw):
    @functools.partial(
        pl.kernel,
        out_type=jax.ShapeDtypeStruct((NC, RN, H), jnp.float32),
        mesh=_MESH,
        scratch_types=[
            pltpu.VMEM_SHARED((RN + TRASH, H), jnp.float32),
            pltpu.VMEM((GSL, EK), jnp.int32),
            pltpu.VMEM((GSL, EK), jnp.int32),
            pltpu.VMEM((EK, H), jnp.float32),
            pltpu.VMEM((EK, H), jnp.float32),
            pltpu.SemaphoreType.DMA,
            pltpu.SemaphoreType.DMA,
            pltpu.SemaphoreType.DMA,
            pltpu.SemaphoreType.DMA,
        ],
        compiler_params=_SC_PARAMS,
    )
    def k(y_hbm, z_hbm, pkg_hbm, pkw_hbm, out_hbm, acc, gslab, wslab,
          rowsa, rowsb, semga, semgb, semsa, semsb):
        cid = lax.axis_index("c")
        sid = lax.axis_index("s")
        wid = sid * NC + cid

        @pl.loop(0, DT)
        def _(t):
            ci = sid + t * NS

            @pl.when(ci < NDC)
            def _():
                pltpu.sync_copy(z_hbm, acc.at[pl.ds(ci * DCH, DCH)])

        plsc.subcore_barrier()

        @pl.loop(0, NT // GSL)
        def _(g):
            pltpu.sync_copy(pkg_hbm.at[pl.ds(g * GSL, GSL), wid], gslab)
            pltpu.sync_copy(pkw_hbm.at[pl.ds(g * GSL, GSL), wid], wslab)
            for j in range(0, GSL, 2):
                ga = pltpu.async_copy(y_hbm.at[gslab.at[j]], rowsa,
                                      semga)
                gb = pltpu.async_copy(y_hbm.at[gslab.at[j + 1]], rowsb,
                                      semgb)
                ga.wait()
                sa = pltpu.async_copy(rowsa, acc.at[wslab.at[j]], semsa,
                                      add=True)
                gb.wait()
                sb = pltpu.async_copy(rowsb, acc.at[wslab.at[j + 1]],
                                      semsb, add=True)
                sa.wait()
                sb.wait()

        plsc.subcore_barrier()

        @pl.loop(0, DT)
        def _(t):
            ci = sid + t * NS

            @pl.when(ci < NDC)
            def _():
                pltpu.sync_copy(acc.at[pl.ds(ci * DCH, DCH)],
                                out_hbm.at[cid, pl.ds(ci * DCH, DCH)])

    return k(y_flat, zeros, pkg, pkw)


def _sc_counts(zeros, pkw):
    @functools.partial(
        pl.kernel,
        out_type=jax.ShapeDtypeStruct((NC, RN, L), jnp.float32),
        mesh=_MESH,
        scratch_types=[
            pltpu.VMEM_SHARED((RN + TRASH, L), jnp.float32),
            pltpu.VMEM((GSL, EK), jnp.int32),
            pltpu.VMEM((EK, L), jnp.float32),
        ],
        compiler_params=_SC_PARAMS,
    )
    def k(z_hbm, pkw_hbm, out_hbm, acc, slab, obuf):
        cid = lax.axis_index("c")
        sid = lax.axis_index("s")
        wid = sid * NC + cid

        onehot = jnp.where(lax.iota(jnp.int32, L) == 0,
                           jnp.float32(1.0), jnp.float32(0.0))

        @pl.loop(0, EK)
        def _(i):
            obuf[i, :] = onehot

        @pl.loop(0, DT)
        def _(t):
            ci = sid + t * NS

            @pl.when(ci < NDC)
            def _():
                pltpu.sync_copy(z_hbm, acc.at[pl.ds(ci * DCH, DCH)])

        plsc.subcore_barrier()

        @pl.loop(0, NT // GSL)
        def _(g):
            pltpu.sync_copy(pkw_hbm.at[pl.ds(g * GSL, GSL), wid], slab)
            for j in range(GSL):
                pltpu.sync_copy(obuf, acc.at[slab.at[j]], add=True)

        plsc.subcore_barrier()

        @pl.loop(0, DT)
        def _(t):
            ci = sid + t * NS

            @pl.when(ci < NDC)
            def _():
                pltpu.sync_copy(acc.at[pl.ds(ci * DCH, DCH)],
                                out_hbm.at[cid, pl.ds(ci * DCH, DCH)])

    return k(zeros, pkw)


_TB = 16


def _tc_pack(src_p, dst_p, typ_p):
    def body(s_ref, d_ref, t_ref, pkg_ref, pkw_ref):
        tn = t_ref[...] * N
        pkg_ref[...] = tn + s_ref[...]
        pkw_ref[...] = tn + d_ref[...]

    return pl.pallas_call(
        body,
        grid=(NT // _TB,),
        in_specs=[
            pl.BlockSpec((_TB, NW, EK), lambda i: (i, 0, 0)),
            pl.BlockSpec((_TB, NW, EK), lambda i: (i, 0, 0)),
            pl.BlockSpec((_TB, NW, EK), lambda i: (i, 0, 0)),
        ],
        out_specs=[
            pl.BlockSpec((_TB, NW, EK), lambda i: (i, 0, 0)),
            pl.BlockSpec((_TB, NW, EK), lambda i: (i, 0, 0)),
        ],
        out_shape=[
            jax.ShapeDtypeStruct((NT, NW, EK), jnp.int32),
            jax.ShapeDtypeStruct((NT, NW, EK), jnp.int32),
        ],
    )(src_p, dst_p, typ_p)


_NB = 2000


def _dot(a, b):
    return jax.lax.dot_general(a, b, (((1,), (0,)), ((), ())),
                               preferred_element_type=jnp.float32)


def _dense1(x, W_rel1, W_root1, b1):
    def body(x_ref, wr_ref, wroot_ref, b_ref, y_ref, root_ref):
        xb = x_ref[...]
        for r in range(R):
            y_ref[r] = _dot(xb, wr_ref[r])
        root_ref[...] = _dot(xb, wroot_ref[...]) + b_ref[...]

    grid = (N // _NB,)
    y, root = pl.pallas_call(
        body,
        grid=grid,
        in_specs=[
            pl.BlockSpec((_NB, IN), lambda i: (i, 0)),
            pl.BlockSpec((R, IN, H), lambda i: (0, 0, 0)),
            pl.BlockSpec((IN, H), lambda i: (0, 0)),
            pl.BlockSpec((1, H), lambda i: (0, 0)),
        ],
        out_specs=[
            pl.BlockSpec((R, _NB, H), lambda i: (0, i, 0)),
            pl.BlockSpec((_NB, H), lambda i: (i, 0)),
        ],
        out_shape=[
            jax.ShapeDtypeStruct((R, N, H), jnp.float32),
            jax.ShapeDtypeStruct((N, H), jnp.float32),
        ],
    )(x, W_rel1, W_root1, b1.reshape(1, H))
    return y.reshape(RN, H), root


def _combine(root_blk, p_ref, cnt_ref):
    h = root_blk
    for r in range(R):
        s = p_ref[0, r] + p_ref[1, r]
        cnt = cnt_ref[0, r, :, 0:1] + cnt_ref[1, r, :, 0:1]
        h = h + s * (1.0 / jnp.maximum(cnt, 1.0))
    return jnp.maximum(h, 0.0)


def _dense2(root1, p1, cnt, W_rel2, W_root2, b2):
    def body(root_ref, p_ref, cnt_ref, wr_ref, wroot_ref, b_ref,
             y_ref, root2_ref):
        h = _combine(root_ref[...], p_ref, cnt_ref)
        for r in range(R):
            y_ref[r] = _dot(h, wr_ref[r])
        root2_ref[...] = _dot(h, wroot_ref[...]) + b_ref[...]

    grid = (N // _NB,)
    y, root2 = pl.pallas_call(
        body,
        grid=grid,
        in_specs=[
            pl.BlockSpec((_NB, H), lambda i: (i, 0)),
            pl.BlockSpec((NC, R, _NB, H), lambda i: (0, 0, i, 0)),
            pl.BlockSpec((NC, R, _NB, L), lambda i: (0, 0, i, 0)),
            pl.BlockSpec((R, H, H), lambda i: (0, 0, 0)),
            pl.BlockSpec((H, H), lambda i: (0, 0)),
            pl.BlockSpec((1, H), lambda i: (0, 0)),
        ],
        out_specs=[
            pl.BlockSpec((R, _NB, H), lambda i: (0, i, 0)),
            pl.BlockSpec((_NB, H), lambda i: (i, 0)),
        ],
        out_shape=[
            jax.ShapeDtypeStruct((R, N, H), jnp.float32),
            jax.ShapeDtypeStruct((N, H), jnp.float32),
        ],
    )(root1, p1.reshape(NC, R, N, H), cnt.reshape(NC, R, N, L),
      W_rel2, W_root2, b2.reshape(1, H))
    return y.reshape(RN, H), root2


def _final(root2, p2, cnt, Wc, bc):
    def body(root_ref, p_ref, cnt_ref, wc_ref, bc_ref, out_ref):
        h = _combine(root_ref[...], p_ref, cnt_ref)
        out_ref[...] = _dot(h, wc_ref[...]) + bc_ref[...]

    grid = (N // _NB,)
    return pl.pallas_call(
        body,
        grid=grid,
        in_specs=[
            pl.BlockSpec((_NB, H), lambda i: (i, 0)),
            pl.BlockSpec((NC, R, _NB, H), lambda i: (0, 0, i, 0)),
            pl.BlockSpec((NC, R, _NB, L), lambda i: (0, 0, i, 0)),
            pl.BlockSpec((H, C), lambda i: (0, 0)),
            pl.BlockSpec((1, C), lambda i: (0, 0)),
        ],
        out_specs=pl.BlockSpec((_NB, C), lambda i: (i, 0)),
        out_shape=jax.ShapeDtypeStruct((N, C), jnp.float32),
    )(root2, p2.reshape(NC, R, N, H), cnt.reshape(NC, R, N, L),
      Wc, bc.reshape(1, C))


def kernel(x, edge_index, edge_type, W_rel1, W_root1, b1,
           W_rel2, W_root2, b2, Wc, bc):
    src = edge_index[0]
    dst = edge_index[1]
    npad = EP - E
    spread = jnp.arange(npad, dtype=jnp.int32)
    src_p = jnp.concatenate(
        [src, spread % N]).reshape(NT, NW, EK)
    dst_p = jnp.concatenate(
        [dst, N + spread % TRASH]).reshape(NT, NW, EK)
    typ_p = jnp.concatenate(
        [edge_type, jnp.full((npad,), R - 1, jnp.int32)]).reshape(
            NT, NW, EK)
    pkg, pkw = _tc_pack(src_p, dst_p, typ_p)
    zeros = jnp.zeros((DCH, H), jnp.float32)
    cnt = _sc_counts(jnp.zeros((DCH, L), jnp.float32), pkw)
    y1, root1 = _dense1(x, W_rel1, W_root1, b1)
    p1 = _sc_aggregate(y1, zeros, pkg, pkw)
    y2, root2 = _dense2(root1, p1, cnt, W_rel2, W_root2, b2)
    p2 = _sc_aggregate(y2, zeros, pkg, pkw)
    return _final(root2, p2, cnt, Wc, bc)
